# Initial kernel scaffold; baseline (speedup 1.0000x reference)
import jax
import jax.numpy as jnp
from jax import lax
from jax.experimental import pallas as pl
from jax.experimental.pallas import tpu as pltpu
from jax.experimental.pallas import tpu_sc as plsc

N = 10000
E = 320000
EP = 327680
IDX_ROWS = EP // 128
ACC_N = 10240
TRASH = 10016
DHID = 256
DOUT = 128
TCOLS = 144
RB = 2000
GRID = N // RB


def _mesh():
    return plsc.VectorSubcoreMesh(core_axis_name="c", subcore_axis_name="s")


def _sc_deg_body(dst_hbm, deg_hbm, dstv, onesv, zerov, acc):
    c = lax.axis_index("c")
    s = lax.axis_index("s")

    @pl.loop(0, 128)
    def _(i):
        onesv[i, :] = jnp.ones((16,), jnp.float32)

    @pl.loop(0, 16)
    def _(i):
        zerov[i, :] = jnp.zeros((16,), jnp.float32)

    @pl.loop(0, 40)
    def _(k):
        pltpu.sync_copy(zerov, acc.at[pl.ds(s * 640 + k * 16, 16)])

    plsc.subcore_barrier()

    base = c * 1280 + s * 80
    pltpu.sync_copy(dst_hbm.at[pl.ds(base, 80)], dstv)

    @pl.loop(0, 80)
    def _(j):
        pltpu.sync_copy(onesv, acc.at[dstv.at[j]], add=True)

    plsc.subcore_barrier()
    pltpu.sync_copy(acc.at[pl.ds(s * 625, 625)], deg_hbm.at[c, pl.ds(s * 625, 625)])


def _sc_deg(dst_p):
    kern = pl.kernel(
        _sc_deg_body,
        out_type=jax.ShapeDtypeStruct((2, N, 16), jnp.float32),
        mesh=_mesh(),
        scratch_types=[
            pltpu.VMEM((80, 128), jnp.int32),
            pltpu.VMEM((128, 16), jnp.float32),
            pltpu.VMEM((16, 16), jnp.float32),
            pltpu.VMEM_SHARED((ACC_N, 16), jnp.float32),
        ],
    )
    return kern(dst_p)


def _sc_gcn_body(g_hbm, src_hbm, dst_hbm, out_hbm,
                 srcv, dstv, rows0, rows1, zerov, acc, sem0, sem1):
    c = lax.axis_index("c")
    s = lax.axis_index("s")

    @pl.loop(0, 16)
    def _(i):
        for k in range(8):
            zerov[i, pl.ds(k * 16, 16)] = jnp.zeros((16,), jnp.float32)

    @pl.loop(0, 40)
    def _(k):
        pltpu.sync_copy(zerov, acc.at[pl.ds(s * 640 + k * 16, 16)])

    base = s * 160
    pltpu.sync_copy(src_hbm.at[pl.ds(base, 160)], srcv)
    pltpu.sync_copy(dst_hbm.at[pl.ds(base, 160)], dstv)

    off = jnp.full((16,), c * N, jnp.int32)

    @pl.loop(0, 160)
    def _(j):
        for k in range(8):
            sl = pl.ds(k * 16, 16)
            srcv[j, sl] = srcv[j, sl] + off

    plsc.subcore_barrier()

    @pl.loop(0, 160, step=2)
    def _(j):
        cp0 = pltpu.make_async_copy(g_hbm.at[srcv.at[j]], rows0, sem0)
        cp1 = pltpu.make_async_copy(g_hbm.at[srcv.at[j + 1]], rows1, sem1)
        cp0.start()
        cp1.start()
        cp0.wait()
        pltpu.sync_copy(rows0, acc.at[dstv.at[j]], add=True)
        cp1.wait()
        pltpu.sync_copy(rows1, acc.at[dstv.at[j + 1]], add=True)

    plsc.subcore_barrier()
    pltpu.sync_copy(acc.at[pl.ds(s * 625, 625)], out_hbm.at[c, pl.ds(s * 625, 625)])


def _sc_gcn(g_tab, src_p, dst_p):
    kern = pl.kernel(
        _sc_gcn_body,
        out_type=jax.ShapeDtypeStruct((2, N, 128), jnp.float32),
        mesh=_mesh(),
        scratch_types=[
            pltpu.VMEM((160, 128), jnp.int32),
            pltpu.VMEM((160, 128), jnp.int32),
            pltpu.VMEM((128, 128), jnp.float32),
            pltpu.VMEM((128, 128), jnp.float32),
            pltpu.VMEM((16, 128), jnp.float32),
            pltpu.VMEM_SHARED((ACC_N, 128), jnp.float32),
            pltpu.SemaphoreType.DMA,
            pltpu.SemaphoreType.DMA,
        ],
    )
    return kern(g_tab, src_p, dst_p)


def _sc_gat_body(t_hbm, src_hbm, dst_hbm, as_hbm, ad_hbm, out_hbm,
                 srcv, dstv, asv, adv, rows0, rows1, zerov, acc, sem0, sem1):
    c = lax.axis_index("c")
    s = lax.axis_index("s")

    @pl.loop(0, 16)
    def _(i):
        for k in range(9):
            zerov[i, pl.ds(k * 16, 16)] = jnp.zeros((16,), jnp.float32)

    @pl.loop(0, 40)
    def _(k):
        pltpu.sync_copy(zerov, acc.at[pl.ds(s * 640 + k * 16, 16)])

    for t in range(15):
        adv[pl.ds(N + t * 16, 16)] = jnp.zeros((16,), jnp.float32)
    pltpu.sync_copy(as_hbm, asv)
    pltpu.sync_copy(ad_hbm, adv.at[pl.ds(0, N)])

    base = s * 160
    pltpu.sync_copy(src_hbm.at[pl.ds(base, 160)], srcv)
    pltpu.sync_copy(dst_hbm.at[pl.ds(base, 160)], dstv)

    off = jnp.full((16,), c * N, jnp.int32)
    cfill = jnp.full((16,), c, jnp.int32)
    trash16 = jnp.full((16,), TRASH, jnp.int32)

    @pl.loop(0, 160)
    def _(j):
        for k in range(8):
            sl = pl.ds(k * 16, 16)
            s16 = srcv[j, sl]
            d16 = dstv[j, sl]
            av = plsc.load_gather(asv, [s16])
            dv = plsc.load_gather(adv, [d16])
            z = av + dv
            zcls = jnp.where(z > 0.0, 0, 1).astype(jnp.int32)
            keep = zcls == cfill
            dstv[j, sl] = jnp.where(keep, d16, trash16)
            srcv[j, sl] = s16 + off

    plsc.subcore_barrier()

    @pl.loop(0, 160, step=2)
    def _(j):
        cp0 = pltpu.make_async_copy(t_hbm.at[srcv.at[j]], rows0, sem0)
        cp1 = pltpu.make_async_copy(t_hbm.at[srcv.at[j + 1]], rows1, sem1)
        cp0.start()
        cp1.start()
        cp0.wait()
        pltpu.sync_copy(rows0, acc.at[dstv.at[j]], add=True)
        cp1.wait()
        pltpu.sync_copy(rows1, acc.at[dstv.at[j + 1]], add=True)

    plsc.subcore_barrier()
    pltpu.sync_copy(acc.at[pl.ds(s * 625, 625)], out_hbm.at[c, pl.ds(s * 625, 625)])


def _sc_gat(t_tab, src_p, dst_p, a_s, a_d):
    kern = pl.kernel(
        _sc_gat_body,
        out_type=jax.ShapeDtypeStruct((2, N, TCOLS), jnp.float32),
        mesh=_mesh(),
        scratch_types=[
            pltpu.VMEM((160, 128), jnp.int32),
            pltpu.VMEM((160, 128), jnp.int32),
            pltpu.VMEM((N,), jnp.float32),
            pltpu.VMEM((ACC_N,), jnp.float32),
            pltpu.VMEM((128, TCOLS), jnp.float32),
            pltpu.VMEM((128, TCOLS), jnp.float32),
            pltpu.VMEM((16, TCOLS), jnp.float32),
            pltpu.VMEM_SHARED((ACC_N, TCOLS), jnp.float32),
            pltpu.SemaphoreType.DMA,
            pltpu.SemaphoreType.DMA,
        ],
    )
    return kern(t_tab, src_p, dst_p, a_s, a_d)


def _mm_body(x_ref, w_ref, o_ref):
    o_ref[...] = lax.dot_general(
        x_ref[...], w_ref[...], (((1,), (0,)), ((), ())),
        precision=lax.Precision.HIGHEST)


def _tc_h(x, w_gcn):
    return pl.pallas_call(
        _mm_body,
        grid=(GRID,),
        in_specs=[pl.BlockSpec((RB, 128), lambda i: (i, 0)),
                  pl.BlockSpec((128, DHID), lambda i: (0, 0))],
        out_specs=pl.BlockSpec((RB, DHID), lambda i: (i, 0)),
        out_shape=jax.ShapeDtypeStruct((N, DHID), jnp.float32),
    )(x, w_gcn)


def _scale_body(deg_ref, h_ref, g_ref):
    deg = deg_ref[0, :, 0] + deg_ref[1, :, 0] + 1.0
    dinv = lax.rsqrt(deg)
    hb = h_ref[...]
    g_ref[0, :, :] = hb[:, :128] * dinv[:, None]
    g_ref[1, :, :] = hb[:, 128:] * dinv[:, None]


def _tc_scale(deg_raw, h):
    return pl.pallas_call(
        _scale_body,
        grid=(GRID,),
        in_specs=[pl.BlockSpec((2, RB, 16), lambda i: (0, i, 0)),
                  pl.BlockSpec((RB, DHID), lambda i: (i, 0))],
        out_specs=pl.BlockSpec((2, RB, 128), lambda i: (0, i, 0)),
        out_shape=jax.ShapeDtypeStruct((2, N, 128), jnp.float32),
    )(deg_raw, h)


def _gcn_fin_body(agg_ref, g_ref, deg_ref, bgcn_ref, wgat_ref, asrc_ref, adst_ref,
                  h2_ref, as_ref, ad_ref, ca_ref, cd_ref):
    i = pl.program_id(0)
    deg = deg_ref[0, :, 0] + deg_ref[1, :, 0] + 1.0
    dinv = lax.rsqrt(deg)
    b = bgcn_ref[...]
    w = wgat_ref[...]
    hg0 = jnp.maximum((agg_ref[0, :, :] + g_ref[0, :, :]) * dinv[:, None]
                      + b[None, :128], 0.0)
    hg1 = jnp.maximum((agg_ref[1, :, :] + g_ref[1, :, :]) * dinv[:, None]
                      + b[None, 128:], 0.0)
    h2 = (lax.dot_general(hg0, w[:128, :], (((1,), (0,)), ((), ())),
                          precision=lax.Precision.HIGHEST)
          + lax.dot_general(hg1, w[128:, :], (((1,), (0,)), ((), ())),
                            precision=lax.Precision.HIGHEST))
    a_s = jnp.sum(h2 * asrc_ref[...][None, :], axis=1)
    a_d = jnp.sum(h2 * adst_ref[...][None, :], axis=1)
    h2_ref[...] = h2
    as_ref[...] = a_s
    ad_ref[...] = a_d

    @pl.when(i == 0)
    def _():
        ca_ref[...] = jnp.full((1, 1), -jnp.inf, jnp.float32)
        cd_ref[...] = jnp.full((1, 1), -jnp.inf, jnp.float32)

    ca_ref[...] = jnp.maximum(ca_ref[...], jnp.full((1, 1), jnp.max(a_s)))
    cd_ref[...] = jnp.maximum(cd_ref[...], jnp.full((1, 1), jnp.max(a_d)))


def _tc_gcn_finish(agg, gsplit2, deg_raw, b_gcn, w_gat, att_src, att_dst):
    return pl.pallas_call(
        _gcn_fin_body,
        grid=(GRID,),
        in_specs=[pl.BlockSpec((2, RB, 128), lambda i: (0, i, 0)),
                  pl.BlockSpec((2, RB, 128), lambda i: (0, i, 0)),
                  pl.BlockSpec((2, RB, 16), lambda i: (0, i, 0)),
                  pl.BlockSpec((DHID,), lambda i: (0,)),
                  pl.BlockSpec((DHID, DOUT), lambda i: (0, 0)),
                  pl.BlockSpec((DOUT,), lambda i: (0,)),
                  pl.BlockSpec((DOUT,), lambda i: (0,))],
        out_specs=[pl.BlockSpec((RB, DOUT), lambda i: (i, 0)),
                   pl.BlockSpec((RB,), lambda i: (i,)),
                   pl.BlockSpec((RB,), lambda i: (i,)),
                   pl.BlockSpec((1, 1), lambda i: (0, 0)),
                   pl.BlockSpec((1, 1), lambda i: (0, 0))],
        out_shape=[jax.ShapeDtypeStruct((N, DOUT), jnp.float32),
                   jax.ShapeDtypeStruct((N,), jnp.float32),
                   jax.ShapeDtypeStruct((N,), jnp.float32),
                   jax.ShapeDtypeStruct((1, 1), jnp.float32),
                   jax.ShapeDtypeStruct((1, 1), jnp.float32)],
    )(agg, gsplit2, deg_raw, b_gcn, w_gat, att_src, att_dst)


def _build_t_body(h2_ref, as_ref, ad_ref, ca_ref, cd_ref, t_ref):
    a = as_ref[...]
    d = ad_ref[...]
    ca = ca_ref[...][0, 0]
    cd = cd_ref[...][0, 0]
    h2 = h2_ref[...]
    f = jnp.exp(a - ca)
    f2 = jnp.exp(0.2 * a - ca)
    gd = jnp.exp(d - cd)
    g2d = jnp.exp(0.2 * d - cd)
    zs = a + d
    ws = jnp.exp(jnp.where(zs > 0.0, zs, 0.2 * zs) - ca - cd)
    zpad = jnp.zeros((a.shape[0], TCOLS - 131), jnp.float32)
    t_ref[0, :, :] = jnp.concatenate(
        [f[:, None] * h2, f[:, None], gd[:, None], ws[:, None], zpad], axis=1)
    t_ref[1, :, :] = jnp.concatenate(
        [f2[:, None] * h2, f2[:, None], g2d[:, None], jnp.zeros_like(ws)[:, None],
         zpad], axis=1)


def _tc_build_t(h2, a_s, a_d, ca, cd):
    return pl.pallas_call(
        _build_t_body,
        grid=(GRID,),
        in_specs=[pl.BlockSpec((RB, DOUT), lambda i: (i, 0)),
                  pl.BlockSpec((RB,), lambda i: (i,)),
                  pl.BlockSpec((RB,), lambda i: (i,)),
                  pl.BlockSpec((1, 1), lambda i: (0, 0)),
                  pl.BlockSpec((1, 1), lambda i: (0, 0))],
        out_specs=pl.BlockSpec((2, RB, TCOLS), lambda i: (0, i, 0)),
        out_shape=jax.ShapeDtypeStruct((2, N, TCOLS), jnp.float32),
    )(h2, a_s, a_d, ca, cd)


def _final_body(acc_ref, t_ref, h2_ref, bgat_ref, o_ref):
    A = acc_ref[...]
    T = t_ref[...]
    h2 = h2_ref[...]
    gd = T[0, :, 129]
    g2d = T[1, :, 129]
    ws = T[0, :, 130]
    num = (A[0, :, :128] * gd[:, None] + A[1, :, :128] * g2d[:, None]
           + h2 * ws[:, None])
    den = A[0, :, 128] * gd + A[1, :, 128] * g2d + ws
    o_ref[...] = num / den[:, None] + bgat_ref[...][None, :]


def _tc_final(accg, t2, h2, b_gat):
    return pl.pallas_call(
        _final_body,
        grid=(GRID,),
        in_specs=[pl.BlockSpec((2, RB, TCOLS), lambda i: (0, i, 0)),
                  pl.BlockSpec((2, RB, TCOLS), lambda i: (0, i, 0)),
                  pl.BlockSpec((RB, DOUT), lambda i: (i, 0)),
                  pl.BlockSpec((DOUT,), lambda i: (0,))],
        out_specs=pl.BlockSpec((RB, DOUT), lambda i: (i, 0)),
        out_shape=jax.ShapeDtypeStruct((N, DOUT), jnp.float32),
    )(accg, t2, h2, b_gat)


def kernel(x, edge_index, W_gcn, b_gcn, W_gat, att_src, att_dst, b_gat):
    src = edge_index[0]
    dst = edge_index[1]
    pad = EP - E
    src_p = jnp.concatenate(
        [src, jnp.zeros((pad,), jnp.int32)]).reshape(IDX_ROWS, 128)
    dst_p = jnp.concatenate(
        [dst, jnp.full((pad,), TRASH, jnp.int32)]).reshape(IDX_ROWS, 128)

    deg_raw = _sc_deg(dst_p)
    h = _tc_h(x, W_gcn)
    gsplit2 = _tc_scale(deg_raw, h)
    agg = _sc_gcn(gsplit2.reshape(2 * N, 128), src_p, dst_p)
    h2, a_s, a_d, ca, cd = _tc_gcn_finish(
        agg, gsplit2, deg_raw, b_gcn, W_gat, att_src, att_dst)
    t2 = _tc_build_t(h2, a_s, a_d, ca, cd)
    accg = _sc_gat(t2.reshape(2 * N, TCOLS), src_p, dst_p, a_s, a_d)
    return _tc_final(accg, t2, h2, b_gat)


# speedup vs baseline: 11.2866x; 11.2866x over previous
import dataclasses

import jax
import jax.numpy as jnp
from jax import lax
from jax.experimental import pallas as pl
from jax.experimental.pallas import tpu as pltpu
from jax.experimental.pallas import tpu_sc as plsc

N = 10000
E = 320000
EP = 327680
IDX_ROWS = EP // 128
ACC_N = 10112
TRASH = 10016
DHID = 256
DOUT = 128
TCOLS = 144
RB = 2000
GRID = N // RB


def _mesh():
    return plsc.VectorSubcoreMesh(core_axis_name="c", subcore_axis_name="s")


def _sc_params():
    cp = pltpu.CompilerParams()
    if "needs_layout_passes" in pltpu.CompilerParams.__dataclass_fields__:
        cp = dataclasses.replace(cp, needs_layout_passes=False)
    return cp


def _sc_deg_body(dst_hbm, deg_hbm, dstv, onesv, zerov, acc):
    c = lax.axis_index("c")
    s = lax.axis_index("s")

    @pl.loop(0, 128)
    def _(i):
        onesv[i, :] = jnp.ones((16,), jnp.float32)

    @pl.loop(0, 8)
    def _(i):
        zerov[i, :] = jnp.zeros((16,), jnp.float32)

    @pl.loop(0, 79)
    def _(k):
        pltpu.sync_copy(zerov, acc.at[pl.ds(s * 632 + k * 8, 8)])

    plsc.subcore_barrier()

    base = c * 1280 + s * 80
    pltpu.sync_copy(dst_hbm.at[pl.ds(base, 80)], dstv)

    @pl.loop(0, 80)
    def _(j):
        pltpu.sync_copy(onesv, acc.at[dstv.at[j]], add=True)

    plsc.subcore_barrier()
    pltpu.sync_copy(acc.at[pl.ds(s * 632, 632)], deg_hbm.at[c, pl.ds(s * 632, 632)])


def _sc_deg(dst_p):
    kern = pl.kernel(
        _sc_deg_body,
        out_type=jax.ShapeDtypeStruct((2, ACC_N, 16), jnp.float32),
        mesh=_mesh(),
        scratch_types=[
            pltpu.VMEM((80, 128), jnp.int32),
            pltpu.VMEM((128, 16), jnp.float32),
            pltpu.VMEM((8, 16), jnp.float32),
            pltpu.VMEM_SHARED((ACC_N, 16), jnp.float32),
        ],
    )
    return kern(dst_p)


def _sc_gcn_body(g_hbm, src_hbm, dst_hbm, out_hbm,
                 srcv, dstv, rows0, rows1, acc, sem0, sem1):
    c = lax.axis_index("c")
    s = lax.axis_index("s")

    @pl.loop(0, 8)
    def _(i):
        for k in range(8):
            rows0[i, pl.ds(k * 16, 16)] = jnp.zeros((16,), jnp.float32)

    @pl.loop(0, 79)
    def _(k):
        pltpu.sync_copy(rows0.at[pl.ds(0, 8)], acc.at[pl.ds(s * 632 + k * 8, 8)])

    plsc.subcore_barrier()

    off = jnp.full((16,), c * N, jnp.int32)
    base = s * 160

    @pl.loop(0, 20)
    def _(t):
        pltpu.sync_copy(src_hbm.at[pl.ds(base + t * 8, 8)], srcv)
        pltpu.sync_copy(dst_hbm.at[pl.ds(base + t * 8, 8)], dstv)

        @pl.loop(0, 8)
        def _(j):
            for k in range(8):
                sl = pl.ds(k * 16, 16)
                srcv[j, sl] = srcv[j, sl] + off

        for p in range(4):
            cp0 = pltpu.make_async_copy(g_hbm.at[srcv.at[2 * p]], rows0, sem0)
            cp1 = pltpu.make_async_copy(g_hbm.at[srcv.at[2 * p + 1]], rows1, sem1)
            cp0.start()
            cp1.start()
            cp0.wait()
            pltpu.sync_copy(rows0, acc.at[dstv.at[2 * p]], add=True)
            cp1.wait()
            pltpu.sync_copy(rows1, acc.at[dstv.at[2 * p + 1]], add=True)

    plsc.subcore_barrier()
    pltpu.sync_copy(acc.at[pl.ds(s * 632, 632)], out_hbm.at[c, pl.ds(s * 632, 632)])


def _sc_gcn(g_tab, src_p, dst_p):
    kern = pl.kernel(
        _sc_gcn_body,
        out_type=jax.ShapeDtypeStruct((2, ACC_N, 128), jnp.float32),
        mesh=_mesh(),
        scratch_types=[
            pltpu.VMEM((8, 128), jnp.int32),
            pltpu.VMEM((8, 128), jnp.int32),
            pltpu.VMEM((128, 128), jnp.float32),
            pltpu.VMEM((128, 128), jnp.float32),
            pltpu.VMEM_SHARED((ACC_N, 128), jnp.float32),
            pltpu.SemaphoreType.DMA,
            pltpu.SemaphoreType.DMA,
        ],
    )
    return kern(g_tab, src_p, dst_p)


def _sc_gat_body(t_hbm, src_hbm, dst_hbm, as_hbm, ad_hbm, ca_hbm,
                 out_hbm, den_hbm,
                 srcv, dstv, asv, adv, denp, rows0, cavv, acc):
    c = lax.axis_index("c")
    s = lax.axis_index("s")

    @pl.loop(0, 8)
    def _(i):
        for k in range(8):
            rows0[i, pl.ds(k * 16, 16)] = jnp.zeros((16,), jnp.float32)

    @pl.loop(0, 79)
    def _(k):
        pltpu.sync_copy(rows0.at[pl.ds(0, 8)], acc.at[pl.ds(s * 632 + k * 8, 8)])

    @pl.loop(0, 632)
    def _(i):
        denp[pl.ds(i * 16, 16)] = jnp.zeros((16,), jnp.float32)

    for t in range(7):
        adv[pl.ds(N + t * 16, 16)] = jnp.zeros((16,), jnp.float32)
    pltpu.sync_copy(as_hbm, asv)
    pltpu.sync_copy(ad_hbm, adv.at[pl.ds(0, N)])
    pltpu.sync_copy(ca_hbm, cavv)

    plsc.subcore_barrier()

    off = jnp.full((16,), c * N, jnp.int32)
    cfill = jnp.full((16,), c, jnp.int32)
    trash16 = jnp.full((16,), TRASH, jnp.int32)
    facv = jnp.where(cfill == 0, jnp.full((16,), 1.0, jnp.float32),
                     jnp.full((16,), 0.2, jnp.float32))
    cavec = cavv[...]
    base = s * 320

    @pl.loop(0, 40)
    def _(t):
        pltpu.sync_copy(src_hbm.at[pl.ds(base + t * 8, 8)], srcv)
        pltpu.sync_copy(dst_hbm.at[pl.ds(base + t * 8, 8)], dstv)

        @pl.loop(0, 8)
        def _(j):
            for k in range(4):
                sl = pl.ds(k * 16, 16)
                s16 = srcv[j, sl]
                d16 = dstv[j, sl]
                av = plsc.load_gather(asv, [s16])
                dv = plsc.load_gather(adv, [d16])
                z = av + dv
                zcls = jnp.where(z > 0.0, 0, 1).astype(jnp.int32)
                keep = zcls == cfill
                dnew = jnp.where(keep, d16, trash16)
                dstv[j, sl] = dnew
                srcv[j, sl] = s16 + off
                wden = jnp.exp(av * facv - cavec)
                plsc.addupdate_scatter(denp, [dnew], wden)

            pltpu.sync_copy(t_hbm.at[srcv.at[j]], rows0.at[pl.ds(0, 64)])
            pltpu.sync_copy(rows0.at[pl.ds(0, 64)], acc.at[dstv.at[j]], add=True)

    plsc.subcore_barrier()
    pltpu.sync_copy(acc.at[pl.ds(s * 632, 632)], out_hbm.at[c, pl.ds(s * 632, 632)])
    pltpu.sync_copy(denp, den_hbm.at[c, s])


def _sc_gat(t_tab, src_p64, dst_p64, a_s, a_d, ca16):
    kern = pl.kernel(
        _sc_gat_body,
        out_type=[jax.ShapeDtypeStruct((2, ACC_N, 128), jnp.float32),
                  jax.ShapeDtypeStruct((2, 16, ACC_N), jnp.float32)],
        mesh=_mesh(),
        compiler_params=_sc_params(),
        scratch_types=[
            pltpu.VMEM((8, 64), jnp.int32),
            pltpu.VMEM((8, 64), jnp.int32),
            pltpu.VMEM((N,), jnp.float32),
            pltpu.VMEM((ACC_N,), jnp.float32),
            pltpu.VMEM((ACC_N,), jnp.float32),
            pltpu.VMEM((64, 128), jnp.float32),
            pltpu.VMEM((16,), jnp.float32),
            pltpu.VMEM_SHARED((ACC_N, 128), jnp.float32),
        ],
    )
    return kern(t_tab, src_p64, dst_p64, a_s, a_d, ca16)


def _mm_body(x_ref, w_ref, o_ref):
    o_ref[...] = lax.dot_general(
        x_ref[...], w_ref[...], (((1,), (0,)), ((), ())),
        precision=lax.Precision.HIGHEST)


def _tc_h(x, w_gcn):
    return pl.pallas_call(
        _mm_body,
        grid=(GRID,),
        in_specs=[pl.BlockSpec((RB, 128), lambda i: (i, 0)),
                  pl.BlockSpec((128, DHID), lambda i: (0, 0))],
        out_specs=pl.BlockSpec((RB, DHID), lambda i: (i, 0)),
        out_shape=jax.ShapeDtypeStruct((N, DHID), jnp.float32),
    )(x, w_gcn)


def _scale_body(deg_ref, h_ref, g_ref):
    deg = deg_ref[0, :, 0] + deg_ref[1, :, 0] + 1.0
    dinv = lax.rsqrt(deg)
    hb = h_ref[...]
    g_ref[0, :, :] = hb[:, :128] * dinv[:, None]
    g_ref[1, :, :] = hb[:, 128:] * dinv[:, None]


def _tc_scale(deg_raw, h):
    return pl.pallas_call(
        _scale_body,
        grid=(GRID,),
        in_specs=[pl.BlockSpec((2, RB, 16), lambda i: (0, i, 0)),
                  pl.BlockSpec((RB, DHID), lambda i: (i, 0))],
        out_specs=pl.BlockSpec((2, RB, 128), lambda i: (0, i, 0)),
        out_shape=jax.ShapeDtypeStruct((2, N, 128), jnp.float32),
    )(deg_raw, h)


def _gcn_fin_body(agg_ref, g_ref, deg_ref, bgcn_ref, wgat_ref, h2_ref):
    D = deg_ref[...]
    deg = D[0, :, 0] + D[1, :, 0] + 1.0
    dinv = lax.rsqrt(deg)
    A = agg_ref[...]
    G = g_ref[...]
    b = bgcn_ref[...]
    w = wgat_ref[...]
    hg0 = jnp.maximum((A[0] + G[0]) * dinv[:, None] + b[None, :128], 0.0)
    hg1 = jnp.maximum((A[1] + G[1]) * dinv[:, None] + b[None, 128:], 0.0)
    h2_ref[...] = (
        lax.dot_general(hg0, w[:128, :], (((1,), (0,)), ((), ())),
                        precision=lax.Precision.HIGHEST)
        + lax.dot_general(hg1, w[128:, :], (((1,), (0,)), ((), ())),
                          precision=lax.Precision.HIGHEST))


def _tc_gcn_finish(agg, gsplit2, deg_raw, b_gcn, w_gat):
    return pl.pallas_call(
        _gcn_fin_body,
        grid=(GRID,),
        in_specs=[pl.BlockSpec((2, RB, 128), lambda i: (0, i, 0)),
                  pl.BlockSpec((2, RB, 128), lambda i: (0, i, 0)),
                  pl.BlockSpec((2, RB, 16), lambda i: (0, i, 0)),
                  pl.BlockSpec((DHID,), lambda i: (0,)),
                  pl.BlockSpec((DHID, DOUT), lambda i: (0, 0))],
        out_specs=pl.BlockSpec((RB, DOUT), lambda i: (i, 0)),
        out_shape=jax.ShapeDtypeStruct((N, DOUT), jnp.float32),
    )(agg, gsplit2, deg_raw, b_gcn, w_gat)


def _att_body(h2_ref, asrc_ref, adst_ref,
              as_ref, ad_ref, fs_ref, aux_ref, ca16_ref):
    h2 = h2_ref[...]
    a_s = jnp.sum(h2 * asrc_ref[...][None, :], axis=1)
    a_d = jnp.sum(h2 * adst_ref[...][None, :], axis=1)
    ca = jnp.max(a_s)
    cd = jnp.max(a_d)
    f = jnp.exp(a_s - ca)
    f2 = jnp.exp(0.2 * a_s - ca)
    gd = jnp.exp(a_d - cd)
    g2d = jnp.exp(0.2 * a_d - cd)
    zs = a_s + a_d
    ws = jnp.exp(jnp.where(zs > 0.0, zs, 0.2 * zs) - ca - cd)
    zpad = jnp.zeros((N, 14), jnp.float32)
    as_ref[...] = a_s
    ad_ref[...] = a_d
    fs_ref[...] = jnp.concatenate([f[:, None], f2[:, None], zpad], axis=1)
    aux_ref[...] = jnp.concatenate(
        [gd[:, None], g2d[:, None], ws[:, None], zpad[:, :13]], axis=1)
    ca16_ref[...] = jnp.full((16,), ca, jnp.float32)


def _tc_att(h2, att_src, att_dst):
    return pl.pallas_call(
        _att_body,
        out_shape=[jax.ShapeDtypeStruct((N,), jnp.float32),
                   jax.ShapeDtypeStruct((N,), jnp.float32),
                   jax.ShapeDtypeStruct((N, 16), jnp.float32),
                   jax.ShapeDtypeStruct((N, 16), jnp.float32),
                   jax.ShapeDtypeStruct((16,), jnp.float32)],
    )(h2, att_src, att_dst)


def _build_t_body(h2_ref, fs_ref, t_ref):
    h2 = h2_ref[...]
    fs = fs_ref[...]
    t_ref[0, :, :] = fs[:, 0][:, None] * h2
    t_ref[1, :, :] = fs[:, 1][:, None] * h2


def _tc_build_t(h2, fs16):
    return pl.pallas_call(
        _build_t_body,
        grid=(GRID,),
        in_specs=[pl.BlockSpec((RB, DOUT), lambda i: (i, 0)),
                  pl.BlockSpec((RB, 16), lambda i: (i, 0))],
        out_specs=pl.BlockSpec((2, RB, DOUT), lambda i: (0, i, 0)),
        out_shape=jax.ShapeDtypeStruct((2, N, DOUT), jnp.float32),
    )(h2, fs16)


def _final_body(acc_ref, den_ref, aux_ref, h2_ref, bgat_ref, o_ref):
    A = acc_ref[...]
    dn = den_ref[...]
    aux = aux_ref[...]
    h2 = h2_ref[...]
    gd = aux[:, 0]
    g2d = aux[:, 1]
    ws = aux[:, 2]
    num = (A[0, :, :] * gd[:, None] + A[1, :, :] * g2d[:, None]
           + h2 * ws[:, None])
    den = jnp.sum(dn[0], axis=1) * gd + jnp.sum(dn[1], axis=1) * g2d + ws
    o_ref[...] = num / den[:, None] + bgat_ref[...][None, :]


def _tc_final(accg, den_raw, aux, h2, b_gat):
    return pl.pallas_call(
        _final_body,
        grid=(GRID,),
        in_specs=[pl.BlockSpec((2, RB, DOUT), lambda i: (0, i, 0)),
                  pl.BlockSpec((2, RB, 16), lambda i: (0, i, 0)),
                  pl.BlockSpec((RB, 16), lambda i: (i, 0)),
                  pl.BlockSpec((RB, DOUT), lambda i: (i, 0)),
                  pl.BlockSpec((DOUT,), lambda i: (0,))],
        out_specs=pl.BlockSpec((RB, DOUT), lambda i: (i, 0)),
        out_shape=jax.ShapeDtypeStruct((N, DOUT), jnp.float32),
    )(accg, den_raw, aux, h2, b_gat)


def kernel(x, edge_index, W_gcn, b_gcn, W_gat, att_src, att_dst, b_gat):
    src = edge_index[0]
    dst = edge_index[1]
    pad = EP - E
    src_p = jnp.concatenate(
        [src, jnp.zeros((pad,), jnp.int32)]).reshape(IDX_ROWS, 128)
    dst_p = jnp.concatenate(
        [dst, jnp.full((pad,), TRASH, jnp.int32)]).reshape(IDX_ROWS, 128)

    deg_raw = _sc_deg(dst_p)
    h = _tc_h(x, W_gcn)
    gsplit2 = _tc_scale(deg_raw, h)
    agg = _sc_gcn(gsplit2.reshape(2 * N, 128), src_p, dst_p)
    h2 = _tc_gcn_finish(agg, gsplit2, deg_raw, b_gcn, W_gat)
    a_s, a_d, fs16, aux, ca16 = _tc_att(h2, att_src, att_dst)
    t2 = _tc_build_t(h2, fs16)
    accg, den_part = _sc_gat(t2.reshape(2 * N, 128), src_p.reshape(5120, 64),
                             dst_p.reshape(5120, 64), a_s, a_d, ca16)
    den_raw = jnp.swapaxes(den_part, 1, 2)
    return _tc_final(accg, den_raw, aux, h2, b_gat)
